# Initial kernel scaffold; baseline (speedup 1.0000x reference)
import jax
import jax.numpy as jnp
from jax.experimental import pallas as pl


def kernel(text, text_length, embeddings):
    raise NotImplementedError("write your pallas kernel here")


# speedup vs baseline: 1.0177x; 1.0177x over previous
import functools

import jax
import jax.numpy as jnp
from jax import lax
from jax.experimental import pallas as pl
from jax.experimental.pallas import tpu as pltpu
from jax.experimental.pallas import tpu_sc as plsc

B = 4096
H = 50
D = 64
NC = 2
NS = 16
L = 16
NW = NC * NS
BPW = B // NW
CE = 2
CHUNK_IDX = CE * H
NCHUNK = BPW // CE
NBUF = 2
INV_H = 1.0 / H


def _sc_body(emb_hbm, idx_hbm, out_hbm, idx_v, rows_v, out_v, sems):
  wid = lax.axis_index("s") * NC + lax.axis_index("c")
  base = wid * BPW

  pltpu.sync_copy(idx_hbm.at[wid], idx_v)

  def issue(c, buf):
    return pltpu.async_copy(emb_hbm.at[idx_v.at[c]], rows_v.at[buf],
                            sems.at[buf])

  for b in range(NBUF):
    issue(b, b)

  def accum_elem(rows_buf, e, r):
    def body(j, acc):
      row = e * H + j
      return tuple(
          acc[k] + rows_buf[row, pl.ds(k * L, L)] for k in range(D // L))
    acc = tuple(jnp.zeros((L,), jnp.float32) for _ in range(D // L))
    acc = lax.fori_loop(0, H, body, acc, unroll=5)
    for k in range(D // L):
      out_v[r, pl.ds(k * L, L)] = acc[k] * INV_H

  def chunk_step(g, carry):
    for b in range(NBUF):
      c = g + b
      pltpu.make_async_copy(emb_hbm.at[idx_v.at[c]], rows_v.at[b],
                            sems.at[b]).wait()
      nxt = c + NBUF

      @pl.when(nxt < NCHUNK)
      def _issue_next():
        issue(nxt, b)

      for e in range(CE):
        accum_elem(rows_v.at[b], e, c * CE + e)
    return carry

  lax.fori_loop(0, NCHUNK // NBUF, lambda g, x: chunk_step(g * NBUF, x), 0)

  pltpu.sync_copy(out_v, out_hbm.at[pl.ds(base, BPW)])


@jax.jit
def _sc_pool(embeddings, idx):
  mesh = plsc.VectorSubcoreMesh(core_axis_name="c", subcore_axis_name="s")
  return pl.kernel(
      _sc_body,
      out_type=jax.ShapeDtypeStruct((B, D), jnp.float32),
      mesh=mesh,
      scratch_types=[
          pltpu.VMEM((NCHUNK, CHUNK_IDX), jnp.int32),
          pltpu.VMEM((NBUF, CHUNK_IDX, D), jnp.float32),
          pltpu.VMEM((BPW, D), jnp.float32),
          pltpu.SemaphoreType.DMA((NBUF,)),
      ],
      compiler_params=pltpu.CompilerParams(use_tc_tiling_on_sc=False),
  )(embeddings, idx)


def kernel(text, text_length, embeddings):
  del text_length
  idx = text.astype(jnp.int32).reshape(NW, NCHUNK, CHUNK_IDX)
  return _sc_pool(embeddings, idx)
